# baseline (speedup 1.0000x reference)
import functools

import jax
import jax.numpy as jnp
from jax import lax
from jax.experimental import pallas as pl
from jax.experimental.pallas import tpu as pltpu
from jax.experimental.pallas import tpu_sc as plsc

VOCAB = 1000000
DIM = 64

NC = 2
NS = 16
NW = NC * NS

CHUNK = 128


def _make_lookup(n_idx: int):
    n_rows = n_idx // CHUNK
    rows_per_w = n_rows // NW

    mesh = plsc.VectorSubcoreMesh(core_axis_name="c", subcore_axis_name="s")

    @functools.partial(
        pl.kernel,
        out_type=jax.ShapeDtypeStruct((n_idx, DIM), jnp.float32),
        mesh=mesh,
        scratch_types=[
            pltpu.VMEM((rows_per_w, CHUNK), jnp.int32),
            pltpu.VMEM((CHUNK, DIM), jnp.float32),
            pltpu.SemaphoreType.DMA,
        ],
        compiler_params=pltpu.CompilerParams(use_tc_tiling_on_sc=False),
    )
    def lookup(table_hbm, idx_hbm, out_hbm, idx_v, rows_v, sem):
        wid = lax.axis_index("s") * NC + lax.axis_index("c")
        base_row = wid * rows_per_w
        pltpu.sync_copy(idx_hbm.at[pl.ds(base_row, rows_per_w)], idx_v)

        @pl.loop(0, rows_per_w)
        def body(g):
            pltpu.async_copy(table_hbm.at[idx_v.at[g]], rows_v, sem).wait()
            pltpu.sync_copy(
                rows_v, out_hbm.at[pl.ds((base_row + g) * CHUNK, CHUNK)]
            )

    return lookup


def kernel(input, weight):
    b, l = input.shape
    n_idx = b * l
    idx2d = input.reshape(n_idx // CHUNK, CHUNK).astype(jnp.int32)
    out = _make_lookup(n_idx)(weight, idx2d)
    return out.reshape(b, l, DIM)


# speedup vs baseline: 1.1185x; 1.1185x over previous
import functools

import jax
import jax.numpy as jnp
from jax import lax
from jax.experimental import pallas as pl
from jax.experimental.pallas import tpu as pltpu
from jax.experimental.pallas import tpu_sc as plsc

VOCAB = 1000000
DIM = 64

NC = 2
NS = 16
NW = NC * NS

CHUNK = 128
NBUF = 4


def _make_lookup(n_idx: int):
    n_rows = n_idx // CHUNK
    rows_per_w = n_rows // NW
    assert rows_per_w % NBUF == 0 and rows_per_w >= 2 * NBUF

    mesh = plsc.VectorSubcoreMesh(core_axis_name="c", subcore_axis_name="s")

    @functools.partial(
        pl.kernel,
        out_type=jax.ShapeDtypeStruct((n_idx, DIM), jnp.float32),
        mesh=mesh,
        scratch_types=[
            pltpu.VMEM((rows_per_w, CHUNK), jnp.int32),
            [pltpu.VMEM((CHUNK, DIM), jnp.float32) for _ in range(NBUF)],
            [pltpu.SemaphoreType.DMA for _ in range(NBUF)],
            [pltpu.SemaphoreType.DMA for _ in range(NBUF)],
        ],
        compiler_params=pltpu.CompilerParams(use_tc_tiling_on_sc=False),
    )
    def lookup(table_hbm, idx_hbm, out_hbm, idx_v, rows, gsem, wsem):
        wid = lax.axis_index("s") * NC + lax.axis_index("c")
        base_row = wid * rows_per_w
        pltpu.sync_copy(idx_hbm.at[pl.ds(base_row, rows_per_w)], idx_v)

        def gather_start(g, b):
            pltpu.async_copy(table_hbm.at[idx_v.at[g]], rows[b], gsem[b])

        def gather_wait(b):
            pltpu.make_async_copy(
                table_hbm.at[pl.ds(0, CHUNK)], rows[b], gsem[b]
            ).wait()

        def out_slice(g):
            return out_hbm.at[pl.ds((base_row + g) * CHUNK, CHUNK)]

        def writeback_start(g, b):
            return pltpu.async_copy(rows[b], out_slice(g), wsem[b])

        for b in range(NBUF):
            gather_start(b, b)

        @pl.loop(0, rows_per_w - NBUF, step=NBUF)
        def body(g0):
            for b in range(NBUF):
                g = g0 + b
                gather_wait(b)
                writeback_start(g, b).wait()
                gather_start(g + NBUF, b)

        for b in range(NBUF):
            g = rows_per_w - NBUF + b
            gather_wait(b)
            writeback_start(g, b)
        for b in range(NBUF):
            g = rows_per_w - NBUF + b
            pltpu.make_async_copy(rows[b], out_slice(g), wsem[b]).wait()

    return lookup


def kernel(input, weight):
    b, l = input.shape
    n_idx = b * l
    idx2d = input.reshape(n_idx // CHUNK, CHUNK).astype(jnp.int32)
    out = _make_lookup(n_idx)(weight, idx2d)
    return out.reshape(b, l, DIM)
